# baseline (speedup 1.0000x reference)
import functools

import jax
import jax.numpy as jnp
from jax import lax
from jax.experimental import pallas as pl
from jax.experimental.pallas import tpu as pltpu
from jax.experimental.pallas import tpu_sc as plsc

N = 100000
K = 16
F = 32
KP = 15
KPP = 16
EXT = 0.06
E = N * K

D = 40
NC = 2
NS = 16
NW = NC * NS
PER_W = E // NW
CH = 2000
ITERS = PER_W // CH


def _sc_gather(tbl, nbr):
    mesh = plsc.VectorSubcoreMesh(core_axis_name="c", subcore_axis_name="s")

    @functools.partial(
        pl.kernel,
        mesh=mesh,
        out_type=jax.ShapeDtypeStruct((E, D), jnp.float32),
        scratch_types=[
            pltpu.VMEM((CH,), jnp.int32),
            pltpu.VMEM((CH, D), jnp.float32),
            pltpu.SemaphoreType.DMA,
        ],
        compiler_params=pltpu.CompilerParams(use_tc_tiling_on_sc=False),
    )
    def k(tbl_hbm, nbr_hbm, out_hbm, idx_v, rows_v, sem):
        wid = lax.axis_index("s") * NC + lax.axis_index("c")

        def body(i, carry):
            base = wid * PER_W + i * CH
            pltpu.sync_copy(nbr_hbm.at[pl.ds(base, CH)], idx_v)
            pltpu.async_copy(tbl_hbm.at[idx_v], rows_v, sem).wait()
            pltpu.sync_copy(rows_v, out_hbm.at[pl.ds(base, CH)])
            return carry

        lax.fori_loop(0, ITERS, body, 0)

    return k(tbl, nbr)


NB = 200
EB = NB * K
WTOT = KPP * F


def _tc_body(ge_ref, pos_ref, kp_ref, r_ref, wc_ref, h_ref, out_ref):
    xg = ge_ref[:, 0:F]
    posv = pos_ref[...]
    posr = jnp.broadcast_to(posv[:, None, :], (NB, K, 3)).reshape(EB, 3)
    acc = None
    for c in range(3):
        rc = ge_ref[:, F + c:F + c + 1] - posr[:, c:c + 1]
        dc = rc - kp_ref[c:c + 1, :]
        acc = dc * dc if acc is None else acc + dc * dc
    dist = jnp.sqrt(acc + 1e-12)
    infl = jnp.maximum(0.0, 1.0 - dist / EXT)
    zc = jnp.dot(xg, wc_ref[...], preferred_element_type=jnp.float32)
    ir = jnp.dot(infl, r_ref[...], preferred_element_type=jnp.float32)
    u = zc * ir
    kfz = u.reshape(NB, K, WTOT).sum(axis=1)
    out_ref[...] = jnp.dot(kfz, h_ref[...],
                           preferred_element_type=jnp.float32)


def _tc(ge, pos, kp_pad, rmat, wcat, hmat):
    return pl.pallas_call(
        _tc_body,
        grid=(N // NB,),
        in_specs=[
            pl.BlockSpec((EB, D), lambda i: (i, 0)),
            pl.BlockSpec((NB, 3), lambda i: (i, 0)),
            pl.BlockSpec((8, KPP), lambda i: (0, 0)),
            pl.BlockSpec((KPP, WTOT), lambda i: (0, 0)),
            pl.BlockSpec((F, WTOT), lambda i: (0, 0)),
            pl.BlockSpec((WTOT, F), lambda i: (0, 0)),
        ],
        out_specs=pl.BlockSpec((NB, F), lambda i: (i, 0)),
        out_shape=jax.ShapeDtypeStruct((N, F), jnp.float32),
    )(ge, pos, kp_pad, rmat, wcat, hmat)


def kernel(x, pos, neighbors, kernel_points, weights):
    nbr = neighbors.astype(jnp.int32).reshape(E)
    tbl = jnp.concatenate(
        [x, pos, jnp.zeros((N, D - F - 3), jnp.float32)], axis=1)
    g = _sc_gather(tbl, nbr)

    kp_pad = jnp.zeros((8, KPP), jnp.float32).at[0:3, 0:KP].set(
        kernel_points.T)
    rmat = jnp.repeat(jnp.eye(KPP, dtype=jnp.float32), F, axis=1)
    wpad = jnp.concatenate(
        [weights, jnp.zeros((KPP - KP, F, F), jnp.float32)], axis=0)
    wcat = wpad.transpose(1, 0, 2).reshape(F, WTOT)
    hmat = jnp.tile(jnp.eye(F, dtype=jnp.float32), (KPP, 1))
    return _tc(g, pos, kp_pad, rmat, wcat, hmat)
